# baseline (speedup 1.0000x reference)
import jax
import jax.numpy as jnp
from jax.experimental import pallas as pl
from jax.experimental.pallas import tpu as pltpu

HIDDEN = 2048
NQ = 16
NKV = 4
DH = 128
G = NQ // NKV
KERNEL_W = 32
STRIDE = 16
WIN = 512
THETA = 500000.0
T = 2048
NUM_C = (T - KERNEL_W) // STRIDE + 1
C_PAD = 128
QT = 256
WCOLS = 3 * QT


def _llama3_inv_freq():
    inv = 1.0 / (THETA ** (jnp.arange(0, DH, 2, dtype=jnp.float32) / DH))
    factor, lo, hi, orig = 8.0, 1.0, 4.0, 8192.0
    wavelen = 2.0 * jnp.pi / inv
    smooth = jnp.clip((orig / wavelen - lo) / (hi - lo), 0.0, 1.0)
    return jnp.where(
        wavelen > orig / lo,
        inv / factor,
        jnp.where(wavelen < orig / hi, inv, (1.0 - smooth) * inv / factor + smooth * inv),
    )


def _prep_body(x_ref, wq_ref, wk_ref, wv_ref, wg_ref, cos_ref, sin_ref, sw_ref,
               q_ref, k_ref, v_ref, g_ref):
    xb = x_ref[...].astype(jnp.bfloat16)
    qp = jnp.dot(xb, wq_ref[...], preferred_element_type=jnp.float32)
    kp = jnp.dot(xb, wk_ref[...], preferred_element_type=jnp.float32)
    vp = jnp.dot(xb, wv_ref[...], preferred_element_type=jnp.float32)
    gp = jnp.dot(xb, wg_ref[...], preferred_element_type=jnp.float32)
    g_ref[...] = jax.nn.sigmoid(gp)
    cos = cos_ref[...]
    sin = sin_ref[...]
    sw = sw_ref[...]
    scale = DH ** -0.5
    for h in range(NQ):
        xh = qp[:, h * DH : (h + 1) * DH]
        xs = jnp.dot(xh.astype(jnp.bfloat16), sw, preferred_element_type=jnp.float32)
        q_ref[h] = ((xh * cos + xs * sin) * scale).astype(jnp.bfloat16)
    for n in range(NKV):
        xh = kp[:, n * DH : (n + 1) * DH]
        xs = jnp.dot(xh.astype(jnp.bfloat16), sw, preferred_element_type=jnp.float32)
        k_ref[n] = (xh * cos + xs * sin).astype(jnp.bfloat16)
        v_ref[n] = vp[:, n * DH : (n + 1) * DH].astype(jnp.bfloat16)


def _prep(x, wq, wk, wv, wg_pad, cos, sin, sw):
    return pl.pallas_call(
        _prep_body,
        grid=(T // QT,),
        in_specs=[
            pl.BlockSpec((QT, HIDDEN), lambda i: (i, 0)),
            pl.BlockSpec((HIDDEN, NQ * DH), lambda i: (0, 0)),
            pl.BlockSpec((HIDDEN, NKV * DH), lambda i: (0, 0)),
            pl.BlockSpec((HIDDEN, NKV * DH), lambda i: (0, 0)),
            pl.BlockSpec((HIDDEN, 128), lambda i: (0, 0)),
            pl.BlockSpec((QT, DH), lambda i: (i, 0)),
            pl.BlockSpec((QT, DH), lambda i: (i, 0)),
            pl.BlockSpec((DH, DH), lambda i: (0, 0)),
        ],
        out_specs=[
            pl.BlockSpec((NQ, QT, DH), lambda i: (0, i, 0)),
            pl.BlockSpec((NKV, QT, DH), lambda i: (0, i, 0)),
            pl.BlockSpec((NKV, QT, DH), lambda i: (0, i, 0)),
            pl.BlockSpec((QT, 128), lambda i: (i, 0)),
        ],
        out_shape=[
            jax.ShapeDtypeStruct((NQ, T, DH), jnp.bfloat16),
            jax.ShapeDtypeStruct((NKV, T, DH), jnp.bfloat16),
            jax.ShapeDtypeStruct((NKV, T, DH), jnp.bfloat16),
            jax.ShapeDtypeStruct((T, 128), jnp.float32),
        ],
    )(x, wq, wk, wv, wg_pad, cos, sin, sw)


NT = T // QT


def _attn_body_for(base):
    KW = (base + 2) * QT
    WS = max(base - 2, 0) * QT

    def body(q_ref, k_ref, v_ref, ck_ref, cv_ref, g_ref, sel_ref, o_ref):
        it = pl.program_id(0)
        qb = q_ref[0]
        kb = k_ref[0]
        rows = jax.lax.broadcasted_iota(jnp.int32, (QT, KW), 0) + (base + it) * QT
        cols = jax.lax.broadcasted_iota(jnp.int32, (QT, KW), 1)

        s = jax.lax.dot_general(
            qb, kb, (((1,), (1,)), ((), ())), preferred_element_type=jnp.float32
        )
        s = jnp.where(rows >= cols, s, jnp.float32(-1e9))
        mx = jnp.max(s, axis=-1, keepdims=True)
        e = jnp.exp(s - mx)
        l_s = jnp.sum(e, axis=-1, keepdims=True)
        out_s = jnp.dot(
            e.astype(jnp.bfloat16), v_ref[0], preferred_element_type=jnp.float32
        ) / l_s

        ew = e[:, WS:]
        wrows = rows[:, WS:]
        wcols = cols[:, WS:]
        ew = jnp.where((wrows - wcols) < WIN, ew, jnp.float32(0.0))
        l_w = jnp.sum(ew, axis=-1, keepdims=True)
        vw = v_ref[0, WS:KW, :]
        out_w = jnp.dot(
            ew.astype(jnp.bfloat16), vw, preferred_element_type=jnp.float32
        ) / l_w

        ccols = jax.lax.broadcasted_iota(jnp.int32, (QT, C_PAD), 1)
        crows = jax.lax.broadcasted_iota(jnp.int32, (QT, C_PAD), 0) + (base + it) * QT
        cmask = (crows >= ccols * STRIDE + KERNEL_W - 1) & (ccols < NUM_C)
        s_c = jax.lax.dot_general(
            qb, ck_ref[0], (((1,), (1,)), ((), ())), preferred_element_type=jnp.float32
        )
        s_c = jnp.where(cmask, s_c, jnp.float32(-1e9))
        mc = jnp.max(s_c, axis=-1, keepdims=True)
        ec = jnp.exp(s_c - mc)
        l_c = jnp.sum(ec, axis=-1, keepdims=True)
        out_c = jnp.dot(
            ec.astype(jnp.bfloat16), cv_ref[0], preferred_element_type=jnp.float32
        )
        if base == 0:
            valid = (crows[:, :1] >= (KERNEL_W - 1)).astype(jnp.float32)
            out_c = out_c * (valid / l_c)
        else:
            out_c = out_c / l_c

        gsel = jnp.dot(
            g_ref[...].astype(jnp.bfloat16), sel_ref[0], preferred_element_type=jnp.float32
        )
        g0 = gsel[:, 0:1]
        g1 = gsel[:, 1:2]
        g2 = gsel[:, 2:3]
        o_ref[...] = (g0 * out_c + g1 * out_s + g2 * out_w).astype(jnp.bfloat16)

    return body


def _attention(q, k, v, ck, cv, gsig, sel_g):
    outs = []
    for base in range(0, NT, 2):
        KW = (base + 2) * QT
        outs.append(
            pl.pallas_call(
                _attn_body_for(base),
                grid=(2, NQ),
                in_specs=[
                    pl.BlockSpec((1, QT, DH), lambda it, h, base=base: (h, base + it, 0)),
                    pl.BlockSpec((1, KW, DH), lambda it, h: (h // G, 0, 0)),
                    pl.BlockSpec((1, KW, DH), lambda it, h: (h // G, 0, 0)),
                    pl.BlockSpec((1, C_PAD, DH), lambda it, h: (h // G, 0, 0)),
                    pl.BlockSpec((1, C_PAD, DH), lambda it, h: (h // G, 0, 0)),
                    pl.BlockSpec((QT, 128), lambda it, h, base=base: (base + it, 0)),
                    pl.BlockSpec((1, 128, 128), lambda it, h: (h, 0, 0)),
                ],
                out_specs=pl.BlockSpec((QT, DH), lambda it, h: (it, h)),
                out_shape=jax.ShapeDtypeStruct((2 * QT, NQ * DH), jnp.bfloat16),
            )(q, k, v, ck, cv, gsig, sel_g)
        )
    return jnp.concatenate(outs, axis=0)


def _pool_body(pk_ref, pv_ref, k_ref, v_ref, ck_ref, cv_ref):
    ck_ref[0] = jnp.dot(pk_ref[...], k_ref[0], preferred_element_type=jnp.float32).astype(
        jnp.bfloat16
    )
    cv_ref[0] = jnp.dot(pv_ref[...], v_ref[0], preferred_element_type=jnp.float32).astype(
        jnp.bfloat16
    )


def _pool(pool_k, pool_v, k, v):
    return pl.pallas_call(
        _pool_body,
        grid=(NKV,),
        in_specs=[
            pl.BlockSpec((C_PAD, T), lambda n: (0, 0)),
            pl.BlockSpec((C_PAD, T), lambda n: (0, 0)),
            pl.BlockSpec((1, T, DH), lambda n: (n, 0, 0)),
            pl.BlockSpec((1, T, DH), lambda n: (n, 0, 0)),
        ],
        out_specs=[
            pl.BlockSpec((1, C_PAD, DH), lambda n: (n, 0, 0)),
            pl.BlockSpec((1, C_PAD, DH), lambda n: (n, 0, 0)),
        ],
        out_shape=[
            jax.ShapeDtypeStruct((NKV, C_PAD, DH), jnp.bfloat16),
            jax.ShapeDtypeStruct((NKV, C_PAD, DH), jnp.bfloat16),
        ],
    )(pool_k, pool_v, k, v)


def _mm_body(x_ref, w_ref, o_ref):
    o_ref[...] = jnp.dot(x_ref[...], w_ref[...], preferred_element_type=jnp.float32)


def _matmul(x, w, bn):
    M, K = x.shape
    _, N = w.shape
    return pl.pallas_call(
        _mm_body,
        grid=(N // bn,),
        in_specs=[
            pl.BlockSpec((M, K), lambda j: (0, 0)),
            pl.BlockSpec((K, bn), lambda j: (0, j)),
        ],
        out_specs=pl.BlockSpec((M, bn), lambda j: (0, j)),
        out_shape=jax.ShapeDtypeStruct((M, N), jnp.float32),
    )(x, w)


def kernel(hidden_states, Wq, Wk, Wv, Wo, Wg, w_ck, w_cv):
    B, S, H = hidden_states.shape
    x = hidden_states.reshape(B * S, H)

    wq_b = Wq.astype(jnp.bfloat16)
    wk_b = Wk.astype(jnp.bfloat16)
    wv_b = Wv.astype(jnp.bfloat16)
    wg_pad = jnp.pad(Wg, ((0, 0), (0, 128 - NQ * 3))).astype(jnp.bfloat16)

    pos = jnp.arange(T, dtype=jnp.float32)
    f = pos[:, None] * _llama3_inv_freq()[None, :]
    cos = jnp.repeat(jnp.cos(f), 2, axis=1)
    sin = jnp.repeat(jnp.sin(f), 2, axis=1)
    r_ = jnp.arange(DH)[:, None]
    c_ = jnp.arange(DH)[None, :]
    sw = (
        jnp.where((r_ == c_ + 1) & (c_ % 2 == 0), -1.0, 0.0)
        + jnp.where((c_ == r_ + 1) & (r_ % 2 == 0), 1.0, 0.0)
    ).astype(jnp.bfloat16)

    qh, kh, vh, gsig = _prep(x, wq_b, wk_b, wv_b, wg_pad, cos, sin, sw)

    wk_c = jax.nn.softmax(w_ck)
    wv_c = jax.nn.softmax(w_cv)
    cpos = jnp.arange(C_PAD)[:, None]
    tpos = jnp.arange(T)[None, :]
    dlt = tpos - cpos * STRIDE
    live = cpos < NUM_C
    oh = (dlt[None, :, :] == jnp.arange(KERNEL_W)[:, None, None]) & live[None, :, :]
    ohf = oh.astype(jnp.float32)
    pool_k = jnp.einsum("jct,j->ct", ohf, wk_c)
    pool_v = jnp.einsum("jct,j->ct", ohf, wv_c)
    ck, cv = _pool(pool_k.astype(jnp.bfloat16), pool_v.astype(jnp.bfloat16), kh, vh)

    h_ = jnp.arange(NQ)[:, None, None]
    rr = jnp.arange(128)[None, :, None]
    cc = jnp.arange(128)[None, None, :]
    sel_g = ((rr == 3 * h_ + cc) & (cc < 3)).astype(jnp.bfloat16)

    out = _attention(qh, kh, vh, ck, cv, gsig, sel_g)
    y = _matmul(out, Wo.astype(jnp.bfloat16), bn=256)
    return y.reshape(B, S, H)


# speedup vs baseline: 1.0173x; 1.0173x over previous
import jax
import jax.numpy as jnp
from jax.experimental import pallas as pl
from jax.experimental.pallas import tpu as pltpu

HIDDEN = 2048
NQ = 16
NKV = 4
DH = 128
G = NQ // NKV
KERNEL_W = 32
STRIDE = 16
WIN = 512
THETA = 500000.0
T = 2048
NUM_C = (T - KERNEL_W) // STRIDE + 1
C_PAD = 128
QT = 256
WCOLS = 3 * QT


def _llama3_inv_freq():
    inv = 1.0 / (THETA ** (jnp.arange(0, DH, 2, dtype=jnp.float32) / DH))
    factor, lo, hi, orig = 8.0, 1.0, 4.0, 8192.0
    wavelen = 2.0 * jnp.pi / inv
    smooth = jnp.clip((orig / wavelen - lo) / (hi - lo), 0.0, 1.0)
    return jnp.where(
        wavelen > orig / lo,
        inv / factor,
        jnp.where(wavelen < orig / hi, inv, (1.0 - smooth) * inv / factor + smooth * inv),
    )


def _prep_body(x_ref, wq_ref, wk_ref, wv_ref, wg_ref, cos_ref, sin_ref, sw_ref,
               pk_ref, pv_ref, q_ref, k_ref, v_ref, g_ref, ck_ref, cv_ref):
    i = pl.program_id(0)
    xb = x_ref[...].astype(jnp.bfloat16)
    qp = jnp.dot(xb, wq_ref[...], preferred_element_type=jnp.float32)
    kp = jnp.dot(xb, wk_ref[...], preferred_element_type=jnp.float32)
    vp = jnp.dot(xb, wv_ref[...], preferred_element_type=jnp.float32)
    gp = jnp.dot(xb, wg_ref[...], preferred_element_type=jnp.float32)
    g_ref[...] = jax.nn.sigmoid(gp)
    cos = cos_ref[...]
    sin = sin_ref[...]
    sw = sw_ref[...]
    scale = DH ** -0.5
    for h in range(NQ):
        xh = qp[:, h * DH : (h + 1) * DH]
        xs = jnp.dot(xh.astype(jnp.bfloat16), sw, preferred_element_type=jnp.float32)
        q_ref[h] = ((xh * cos + xs * sin) * scale).astype(jnp.bfloat16)
    for n in range(NKV):
        xh = kp[:, n * DH : (n + 1) * DH]
        xs = jnp.dot(xh.astype(jnp.bfloat16), sw, preferred_element_type=jnp.float32)
        k_ref[n, pl.ds(i * QT, QT), :] = (xh * cos + xs * sin).astype(jnp.bfloat16)
        v_ref[n, pl.ds(i * QT, QT), :] = vp[:, n * DH : (n + 1) * DH].astype(jnp.bfloat16)

    @pl.when(i == T // QT - 1)
    def _():
        for n in range(NKV):
            ck_ref[n] = jnp.dot(
                pk_ref[...], k_ref[n], preferred_element_type=jnp.float32
            ).astype(jnp.bfloat16)
            cv_ref[n] = jnp.dot(
                pv_ref[...], v_ref[n], preferred_element_type=jnp.float32
            ).astype(jnp.bfloat16)


def _prep(x, wq, wk, wv, wg_pad, cos, sin, sw, pool_k, pool_v):
    return pl.pallas_call(
        _prep_body,
        grid=(T // QT,),
        in_specs=[
            pl.BlockSpec((QT, HIDDEN), lambda i: (i, 0)),
            pl.BlockSpec((HIDDEN, NQ * DH), lambda i: (0, 1)),
            pl.BlockSpec((HIDDEN, NKV * DH), lambda i: (0, 8)),
            pl.BlockSpec((HIDDEN, NKV * DH), lambda i: (0, 9)),
            pl.BlockSpec((HIDDEN, 128), lambda i: (0, 40)),
            pl.BlockSpec((QT, DH), lambda i: (i, 0)),
            pl.BlockSpec((QT, DH), lambda i: (i, 0)),
            pl.BlockSpec((DH, DH), lambda i: (0, 0)),
            pl.BlockSpec((C_PAD, T), lambda i: (0, 0)),
            pl.BlockSpec((C_PAD, T), lambda i: (0, 0)),
        ],
        out_specs=[
            pl.BlockSpec((NQ, QT, DH), lambda i: (0, i, 0)),
            pl.BlockSpec((NKV, T, DH), lambda i: (0, 0, 0)),
            pl.BlockSpec((NKV, T, DH), lambda i: (0, 0, 0)),
            pl.BlockSpec((QT, 128), lambda i: (i, 0)),
            pl.BlockSpec((NKV, C_PAD, DH), lambda i: (0, 0, 0)),
            pl.BlockSpec((NKV, C_PAD, DH), lambda i: (0, 0, 0)),
        ],
        out_shape=[
            jax.ShapeDtypeStruct((NQ, T, DH), jnp.bfloat16),
            jax.ShapeDtypeStruct((NKV, T, DH), jnp.bfloat16),
            jax.ShapeDtypeStruct((NKV, T, DH), jnp.bfloat16),
            jax.ShapeDtypeStruct((T, 128), jnp.float32),
            jax.ShapeDtypeStruct((NKV, C_PAD, DH), jnp.bfloat16),
            jax.ShapeDtypeStruct((NKV, C_PAD, DH), jnp.bfloat16),
        ],
    )(x, wq, wk, wv, wg_pad, cos, sin, sw, pool_k, pool_v)


NT = T // QT


def _attn_body_for(base):
    KW = (base + 2) * QT
    WS = max(base - 2, 0) * QT

    def body(q_ref, k_ref, v_ref, ck_ref, cv_ref, g_ref, sel_ref, o_ref):
        it = pl.program_id(0)
        qb = q_ref[0]
        kb = k_ref[0]
        rows = jax.lax.broadcasted_iota(jnp.int32, (QT, KW), 0) + (base + it) * QT
        cols = jax.lax.broadcasted_iota(jnp.int32, (QT, KW), 1)

        s = jax.lax.dot_general(
            qb, kb, (((1,), (1,)), ((), ())), preferred_element_type=jnp.float32
        )
        s = jnp.where(rows >= cols, s, jnp.float32(-1e9))
        mx = jnp.max(s, axis=-1, keepdims=True)
        e = jnp.exp(s - mx)
        l_s = jnp.sum(e, axis=-1, keepdims=True)
        out_s = jnp.dot(
            e.astype(jnp.bfloat16), v_ref[0], preferred_element_type=jnp.float32
        ) / l_s

        ew = e[:, WS:]
        wrows = rows[:, WS:]
        wcols = cols[:, WS:]
        ew = jnp.where((wrows - wcols) < WIN, ew, jnp.float32(0.0))
        l_w = jnp.sum(ew, axis=-1, keepdims=True)
        vw = v_ref[0, WS:KW, :]
        out_w = jnp.dot(
            ew.astype(jnp.bfloat16), vw, preferred_element_type=jnp.float32
        ) / l_w

        ccols = jax.lax.broadcasted_iota(jnp.int32, (QT, C_PAD), 1)
        crows = jax.lax.broadcasted_iota(jnp.int32, (QT, C_PAD), 0) + (base + it) * QT
        cmask = (crows >= ccols * STRIDE + KERNEL_W - 1) & (ccols < NUM_C)
        s_c = jax.lax.dot_general(
            qb, ck_ref[0], (((1,), (1,)), ((), ())), preferred_element_type=jnp.float32
        )
        s_c = jnp.where(cmask, s_c, jnp.float32(-1e9))
        mc = jnp.max(s_c, axis=-1, keepdims=True)
        ec = jnp.exp(s_c - mc)
        l_c = jnp.sum(ec, axis=-1, keepdims=True)
        out_c = jnp.dot(
            ec.astype(jnp.bfloat16), cv_ref[0], preferred_element_type=jnp.float32
        )
        if base == 0:
            valid = (crows[:, :1] >= (KERNEL_W - 1)).astype(jnp.float32)
            out_c = out_c * (valid / l_c)
        else:
            out_c = out_c / l_c

        gsel = jnp.dot(
            g_ref[...].astype(jnp.bfloat16), sel_ref[0], preferred_element_type=jnp.float32
        )
        g0 = gsel[:, 0:1]
        g1 = gsel[:, 1:2]
        g2 = gsel[:, 2:3]
        o_ref[...] = (g0 * out_c + g1 * out_s + g2 * out_w).astype(jnp.bfloat16)

    return body


def _attention(q, k, v, ck, cv, gsig, sel_g):
    outs = []
    for base in range(0, NT, 2):
        KW = (base + 2) * QT
        outs.append(
            pl.pallas_call(
                _attn_body_for(base),
                grid=(2, NQ),
                in_specs=[
                    pl.BlockSpec((1, QT, DH), lambda it, h, base=base: (h, base + it, 0)),
                    pl.BlockSpec((1, KW, DH), lambda it, h: (h // G, 0, 0)),
                    pl.BlockSpec((1, KW, DH), lambda it, h: (h // G, 0, 0)),
                    pl.BlockSpec((1, C_PAD, DH), lambda it, h: (h // G, 0, 0)),
                    pl.BlockSpec((1, C_PAD, DH), lambda it, h: (h // G, 0, 0)),
                    pl.BlockSpec((QT, 128), lambda it, h, base=base: (base + it, 0)),
                    pl.BlockSpec((1, 128, 128), lambda it, h: (h, 0, 0)),
                ],
                out_specs=pl.BlockSpec((QT, DH), lambda it, h: (it, h)),
                out_shape=jax.ShapeDtypeStruct((2 * QT, NQ * DH), jnp.bfloat16),
            )(q, k, v, ck, cv, gsig, sel_g)
        )
    return jnp.concatenate(outs, axis=0)


def _mm_body(x_ref, w_ref, o_ref):
    o_ref[...] = jnp.dot(x_ref[...], w_ref[...], preferred_element_type=jnp.float32)


def _matmul(x, w, bn, n_out):
    M, K = x.shape
    return pl.pallas_call(
        _mm_body,
        grid=(n_out // bn,),
        in_specs=[
            pl.BlockSpec((M, K), lambda j: (0, 0)),
            pl.BlockSpec((K, bn), lambda j: (0, j)),
        ],
        out_specs=pl.BlockSpec((M, bn), lambda j: (0, j)),
        out_shape=jax.ShapeDtypeStruct((M, n_out), jnp.float32),
    )(x, w)


def kernel(hidden_states, Wq, Wk, Wv, Wo, Wg, w_ck, w_cv):
    B, S, H = hidden_states.shape
    x = hidden_states.reshape(B * S, H)

    wg_pad = jnp.pad(Wg, ((0, 0), (0, 128 - NQ * 3)))
    w_cat = jnp.concatenate([Wo, Wq, Wk, Wv, wg_pad], axis=1).astype(jnp.bfloat16)

    pos = jnp.arange(T, dtype=jnp.float32)
    f = pos[:, None] * _llama3_inv_freq()[None, :]
    cos = jnp.repeat(jnp.cos(f), 2, axis=1)
    sin = jnp.repeat(jnp.sin(f), 2, axis=1)
    r_ = jnp.arange(DH)[:, None]
    c_ = jnp.arange(DH)[None, :]
    sw = (
        jnp.where((r_ == c_ + 1) & (c_ % 2 == 0), -1.0, 0.0)
        + jnp.where((c_ == r_ + 1) & (r_ % 2 == 0), 1.0, 0.0)
    ).astype(jnp.bfloat16)

    wk_c = jax.nn.softmax(w_ck)
    wv_c = jax.nn.softmax(w_cv)
    cpos = jnp.arange(C_PAD)[:, None]
    tpos = jnp.arange(T)[None, :]
    dlt = tpos - cpos * STRIDE
    live = cpos < NUM_C
    oh = (dlt[None, :, :] == jnp.arange(KERNEL_W)[:, None, None]) & live[None, :, :]
    ohf = oh.astype(jnp.float32)
    pool_k = jnp.einsum("jct,j->ct", ohf, wk_c).astype(jnp.bfloat16)
    pool_v = jnp.einsum("jct,j->ct", ohf, wv_c).astype(jnp.bfloat16)

    qh, kh, vh, gsig, ck, cv = _prep(
        x, w_cat, w_cat, w_cat, w_cat, cos, sin, sw, pool_k, pool_v
    )

    h_ = jnp.arange(NQ)[:, None, None]
    rr = jnp.arange(128)[None, :, None]
    cc = jnp.arange(128)[None, None, :]
    sel_g = ((rr == 3 * h_ + cc) & (cc < 3)).astype(jnp.bfloat16)

    out = _attention(qh, kh, vh, ck, cv, gsig, sel_g)
    y = _matmul(out, w_cat, bn=256, n_out=HIDDEN)
    return y.reshape(B, S, H)


# speedup vs baseline: 1.0824x; 1.0639x over previous
import jax
import jax.numpy as jnp
from jax.experimental import pallas as pl
from jax.experimental.pallas import tpu as pltpu

HIDDEN = 2048
NQ = 16
NKV = 4
DH = 128
G = NQ // NKV
KERNEL_W = 32
STRIDE = 16
WIN = 512
THETA = 500000.0
T = 2048
NUM_C = (T - KERNEL_W) // STRIDE + 1
C_PAD = 128
QT = 256
WCOLS = 3 * QT


def _llama3_inv_freq():
    inv = 1.0 / (THETA ** (jnp.arange(0, DH, 2, dtype=jnp.float32) / DH))
    factor, lo, hi, orig = 8.0, 1.0, 4.0, 8192.0
    wavelen = 2.0 * jnp.pi / inv
    smooth = jnp.clip((orig / wavelen - lo) / (hi - lo), 0.0, 1.0)
    return jnp.where(
        wavelen > orig / lo,
        inv / factor,
        jnp.where(wavelen < orig / hi, inv, (1.0 - smooth) * inv / factor + smooth * inv),
    )


def _prep_body(x_ref, wq_ref, wk_ref, wv_ref, wg_ref, cos_ref, sin_ref, sw_ref,
               pk_ref, pv_ref, q_ref, k_ref, v_ref, g_ref, ck_ref, cv_ref,
               wqs, wks, wvs, wgs):
    i = pl.program_id(0)

    @pl.when(i == 0)
    def _():
        wqs[...] = wq_ref[...].astype(jnp.bfloat16)
        wks[...] = wk_ref[...].astype(jnp.bfloat16)
        wvs[...] = wv_ref[...].astype(jnp.bfloat16)
        wgs[...] = wg_ref[...].astype(jnp.bfloat16)

    xb = x_ref[...].astype(jnp.bfloat16)
    qp = jnp.dot(xb, wqs[...], preferred_element_type=jnp.float32)
    kp = jnp.dot(xb, wks[...], preferred_element_type=jnp.float32)
    vp = jnp.dot(xb, wvs[...], preferred_element_type=jnp.float32)
    gp = jnp.dot(xb, wgs[...], preferred_element_type=jnp.float32)
    g_ref[...] = jax.nn.sigmoid(gp)
    cos = cos_ref[...]
    sin = sin_ref[...]
    sw = sw_ref[...]
    scale = DH ** -0.5
    for h in range(NQ):
        xh = qp[:, h * DH : (h + 1) * DH]
        xs = jnp.dot(xh.astype(jnp.bfloat16), sw, preferred_element_type=jnp.float32)
        q_ref[h] = ((xh * cos + xs * sin) * scale).astype(jnp.bfloat16)
    for n in range(NKV):
        xh = kp[:, n * DH : (n + 1) * DH]
        xs = jnp.dot(xh.astype(jnp.bfloat16), sw, preferred_element_type=jnp.float32)
        k_ref[n, pl.ds(i * QT, QT), :] = (xh * cos + xs * sin).astype(jnp.bfloat16)
        v_ref[n, pl.ds(i * QT, QT), :] = vp[:, n * DH : (n + 1) * DH].astype(jnp.bfloat16)

    @pl.when(i == T // QT - 1)
    def _():
        for n in range(NKV):
            ck_ref[n] = jnp.dot(
                pk_ref[...], k_ref[n], preferred_element_type=jnp.float32
            ).astype(jnp.bfloat16)
            cv_ref[n] = jnp.dot(
                pv_ref[...], v_ref[n], preferred_element_type=jnp.float32
            ).astype(jnp.bfloat16)


def _prep(x, wq, wk, wv, wg_pad, cos, sin, sw, pool_k, pool_v):
    return pl.pallas_call(
        _prep_body,
        grid=(T // QT,),
        in_specs=[
            pl.BlockSpec((QT, HIDDEN), lambda i: (i, 0)),
            pl.BlockSpec((HIDDEN, NQ * DH), lambda i: (0, 0)),
            pl.BlockSpec((HIDDEN, NKV * DH), lambda i: (0, 0)),
            pl.BlockSpec((HIDDEN, NKV * DH), lambda i: (0, 0)),
            pl.BlockSpec((HIDDEN, 128), lambda i: (0, 0)),
            pl.BlockSpec((QT, DH), lambda i: (i, 0)),
            pl.BlockSpec((QT, DH), lambda i: (i, 0)),
            pl.BlockSpec((DH, DH), lambda i: (0, 0)),
            pl.BlockSpec((C_PAD, T), lambda i: (0, 0)),
            pl.BlockSpec((C_PAD, T), lambda i: (0, 0)),
        ],
        out_specs=[
            pl.BlockSpec((NQ, QT, DH), lambda i: (0, i, 0)),
            pl.BlockSpec((NKV, T, DH), lambda i: (0, 0, 0)),
            pl.BlockSpec((NKV, T, DH), lambda i: (0, 0, 0)),
            pl.BlockSpec((QT, 128), lambda i: (i, 0)),
            pl.BlockSpec((NKV, C_PAD, DH), lambda i: (0, 0, 0)),
            pl.BlockSpec((NKV, C_PAD, DH), lambda i: (0, 0, 0)),
        ],
        out_shape=[
            jax.ShapeDtypeStruct((NQ, T, DH), jnp.bfloat16),
            jax.ShapeDtypeStruct((NKV, T, DH), jnp.bfloat16),
            jax.ShapeDtypeStruct((NKV, T, DH), jnp.bfloat16),
            jax.ShapeDtypeStruct((T, 128), jnp.float32),
            jax.ShapeDtypeStruct((NKV, C_PAD, DH), jnp.bfloat16),
            jax.ShapeDtypeStruct((NKV, C_PAD, DH), jnp.bfloat16),
        ],
        scratch_shapes=[
            pltpu.VMEM((HIDDEN, NQ * DH), jnp.bfloat16),
            pltpu.VMEM((HIDDEN, NKV * DH), jnp.bfloat16),
            pltpu.VMEM((HIDDEN, NKV * DH), jnp.bfloat16),
            pltpu.VMEM((HIDDEN, 128), jnp.bfloat16),
        ],
    )(x, wq, wk, wv, wg_pad, cos, sin, sw, pool_k, pool_v)


NT = T // QT


def _attn_body_for(base):
    KW = (base + 2) * QT
    WS = max(base - 2, 0) * QT

    def body(q_ref, k_ref, v_ref, ck_ref, cv_ref, g_ref, sel_ref, o_ref):
        it = pl.program_id(0)
        qb = q_ref[0]
        kb = k_ref[0]
        rows = jax.lax.broadcasted_iota(jnp.int32, (QT, KW), 0) + (base + it) * QT
        cols = jax.lax.broadcasted_iota(jnp.int32, (QT, KW), 1)

        s = jax.lax.dot_general(
            qb, kb, (((1,), (1,)), ((), ())), preferred_element_type=jnp.float32
        )
        s = jnp.where(rows >= cols, s, jnp.float32(-1e9))
        mx = jnp.max(s, axis=-1, keepdims=True)
        e = jnp.exp(s - mx)
        l_s = jnp.sum(e, axis=-1, keepdims=True)
        out_s = jnp.dot(
            e.astype(jnp.bfloat16), v_ref[0], preferred_element_type=jnp.float32
        ) / l_s

        ew = e[:, WS:]
        wrows = rows[:, WS:]
        wcols = cols[:, WS:]
        ew = jnp.where((wrows - wcols) < WIN, ew, jnp.float32(0.0))
        l_w = jnp.sum(ew, axis=-1, keepdims=True)
        vw = v_ref[0, WS:KW, :]
        out_w = jnp.dot(
            ew.astype(jnp.bfloat16), vw, preferred_element_type=jnp.float32
        ) / l_w

        ccols = jax.lax.broadcasted_iota(jnp.int32, (QT, C_PAD), 1)
        crows = jax.lax.broadcasted_iota(jnp.int32, (QT, C_PAD), 0) + (base + it) * QT
        cmask = (crows >= ccols * STRIDE + KERNEL_W - 1) & (ccols < NUM_C)
        s_c = jax.lax.dot_general(
            qb, ck_ref[0], (((1,), (1,)), ((), ())), preferred_element_type=jnp.float32
        )
        s_c = jnp.where(cmask, s_c, jnp.float32(-1e9))
        mc = jnp.max(s_c, axis=-1, keepdims=True)
        ec = jnp.exp(s_c - mc)
        l_c = jnp.sum(ec, axis=-1, keepdims=True)
        out_c = jnp.dot(
            ec.astype(jnp.bfloat16), cv_ref[0], preferred_element_type=jnp.float32
        )
        if base == 0:
            valid = (crows[:, :1] >= (KERNEL_W - 1)).astype(jnp.float32)
            out_c = out_c * (valid / l_c)
        else:
            out_c = out_c / l_c

        gsel = jnp.dot(
            g_ref[...].astype(jnp.bfloat16), sel_ref[0], preferred_element_type=jnp.float32
        )
        g0 = gsel[:, 0:1]
        g1 = gsel[:, 1:2]
        g2 = gsel[:, 2:3]
        o_ref[...] = (g0 * out_c + g1 * out_s + g2 * out_w).astype(jnp.bfloat16)

    return body


def _attention(q, k, v, ck, cv, gsig, sel_g):
    outs = []
    for base in range(0, NT, 2):
        KW = (base + 2) * QT
        outs.append(
            pl.pallas_call(
                _attn_body_for(base),
                grid=(2, NQ),
                in_specs=[
                    pl.BlockSpec((1, QT, DH), lambda it, h, base=base: (h, base + it, 0)),
                    pl.BlockSpec((1, KW, DH), lambda it, h: (h // G, 0, 0)),
                    pl.BlockSpec((1, KW, DH), lambda it, h: (h // G, 0, 0)),
                    pl.BlockSpec((1, C_PAD, DH), lambda it, h: (h // G, 0, 0)),
                    pl.BlockSpec((1, C_PAD, DH), lambda it, h: (h // G, 0, 0)),
                    pl.BlockSpec((QT, 128), lambda it, h, base=base: (base + it, 0)),
                    pl.BlockSpec((1, 128, 128), lambda it, h: (h, 0, 0)),
                ],
                out_specs=pl.BlockSpec((QT, DH), lambda it, h: (it, h)),
                out_shape=jax.ShapeDtypeStruct((2 * QT, NQ * DH), jnp.bfloat16),
            )(q, k, v, ck, cv, gsig, sel_g)
        )
    return jnp.concatenate(outs, axis=0)


def _mm_body(x_ref, w_ref, o_ref):
    o_ref[...] = jnp.dot(
        x_ref[...], w_ref[...].astype(jnp.bfloat16), preferred_element_type=jnp.float32
    )


def _matmul(x, w, bn, n_out):
    M, K = x.shape
    return pl.pallas_call(
        _mm_body,
        grid=(n_out // bn,),
        in_specs=[
            pl.BlockSpec((M, K), lambda j: (0, 0)),
            pl.BlockSpec((K, bn), lambda j: (0, j)),
        ],
        out_specs=pl.BlockSpec((M, bn), lambda j: (0, j)),
        out_shape=jax.ShapeDtypeStruct((M, n_out), jnp.float32),
    )(x, w)


def kernel(hidden_states, Wq, Wk, Wv, Wo, Wg, w_ck, w_cv):
    B, S, H = hidden_states.shape
    x = hidden_states.reshape(B * S, H)

    wg_pad = jnp.pad(Wg, ((0, 0), (0, 128 - NQ * 3)))

    pos = jnp.arange(T, dtype=jnp.float32)
    f = pos[:, None] * _llama3_inv_freq()[None, :]
    cos = jnp.repeat(jnp.cos(f), 2, axis=1)
    sin = jnp.repeat(jnp.sin(f), 2, axis=1)
    r_ = jnp.arange(DH)[:, None]
    c_ = jnp.arange(DH)[None, :]
    sw = (
        jnp.where((r_ == c_ + 1) & (c_ % 2 == 0), -1.0, 0.0)
        + jnp.where((c_ == r_ + 1) & (r_ % 2 == 0), 1.0, 0.0)
    ).astype(jnp.bfloat16)

    wk_c = jax.nn.softmax(w_ck)
    wv_c = jax.nn.softmax(w_cv)
    cpos = jnp.arange(C_PAD)[:, None]
    tpos = jnp.arange(T)[None, :]
    dlt = tpos - cpos * STRIDE
    live = cpos < NUM_C
    oh = (dlt[None, :, :] == jnp.arange(KERNEL_W)[:, None, None]) & live[None, :, :]
    ohf = oh.astype(jnp.float32)
    pool_k = jnp.einsum("jct,j->ct", ohf, wk_c).astype(jnp.bfloat16)
    pool_v = jnp.einsum("jct,j->ct", ohf, wv_c).astype(jnp.bfloat16)

    qh, kh, vh, gsig, ck, cv = _prep(
        x, Wq, Wk, Wv, wg_pad, cos, sin, sw, pool_k, pool_v
    )

    h_ = jnp.arange(NQ)[:, None, None]
    rr = jnp.arange(128)[None, :, None]
    cc = jnp.arange(128)[None, None, :]
    sel_g = ((rr == 3 * h_ + cc) & (cc < 3)).astype(jnp.bfloat16)

    out = _attention(qh, kh, vh, ck, cv, gsig, sel_g)
    y = _matmul(out, Wo, bn=256, n_out=HIDDEN)
    return y.reshape(B, S, H)
